# Initial kernel scaffold; baseline (speedup 1.0000x reference)
import jax
import jax.numpy as jnp
from jax.experimental import pallas as pl


def kernel(x, pos, edge_index, batch, W1, b1, W2, b2, W3, b3, W4, b4):
    raise NotImplementedError("write your pallas kernel here")


# speedup vs baseline: 17.5129x; 17.5129x over previous
import functools

import jax
import jax.numpy as jnp
from jax import lax
from jax.experimental import pallas as pl
from jax.experimental.pallas import tpu as pltpu
from jax.experimental.pallas import tpu_sc as plsc

N = 100000
E = 1600000
G = 256
L = 16

NUM_TILES = 32
CHUNK = 1024
KSUB = CHUNK // 128
EPAD = 1605632
TILE_CHUNKS = EPAD // NUM_TILES // CHUNK
EROWS_PER_TILE = EPAD // NUM_TILES // 128

NACC = 101120
ROWS_PER_TILE = NACC // 16
ZCH = 316

NPOOL = 102400
POOL_CHUNK = 640
GACC = G + 32

_f32 = jnp.float32


@functools.lru_cache(maxsize=None)
def _mesh():
    return plsc.VectorSubcoreMesh(core_axis_name="c", subcore_axis_name="s")


def _wid(cid, sid):
    return sid * 2 + cid


def _fill_zero(zref, nrows):
    def zrow(i, _):
        zref[i, :] = jnp.zeros((L,), _f32)
        return 0
    lax.fori_loop(0, nrows, zrow, 0)


def _zero_acc(acc, zero_v, sid, rows_per_tile, zch):
    def zchunk(i, _):
        pltpu.sync_copy(zero_v, acc.at[pl.ds(sid * rows_per_tile + i * zch, zch)])
        return 0
    lax.fori_loop(0, rows_per_tile // zch, zchunk, 0)


def _edge_scatter_pass(g_hbm, src2d, dst2d, src_v, dst_v, rows_v, acc, sem, wid):
    def echunk(c, _):
        row0 = wid * EROWS_PER_TILE + c * KSUB
        if g_hbm is not None:
            pltpu.sync_copy(src2d.at[pl.ds(row0, KSUB)], src_v)
        pltpu.sync_copy(dst2d.at[pl.ds(row0, KSUB)], dst_v)
        if g_hbm is not None:
            def gfire(j, _):
                pltpu.async_copy(
                    g_hbm.at[src_v.at[j]], rows_v.at[pl.ds(j * 128, 128)], sem)
                return 0
            lax.fori_loop(0, KSUB, gfire, 0)
            pltpu.make_async_copy(
                g_hbm.at[pl.ds(0, CHUNK)], rows_v, sem).wait()
        def sfire(j, _):
            pltpu.sync_copy(
                rows_v.at[pl.ds(j * 128, 128)], acc.at[dst_v.at[j]], add=True)
            return 0
        lax.fori_loop(0, KSUB, sfire, 0)
        return 0
    lax.fori_loop(0, TILE_CHUNKS, echunk, 0)


def _make_edge_pass(num_slices, gather):
    n_in = (2 + num_slices) if gather else 1

    def body(*refs):
        ins = refs[:n_in]
        outs = refs[n_in:n_in + num_slices]
        src_v, dst_v, rows_v, zero_v, acc, sem = refs[n_in + num_slices:]
        dst2d = ins[0]
        src2d = ins[1] if gather else None
        cid = lax.axis_index("c")
        sid = lax.axis_index("s")
        wid = _wid(cid, sid)
        _fill_zero(zero_v, ZCH)
        if not gather:
            def orow(i, _):
                rows_v[i, :] = jnp.ones((L,), _f32)
                return 0
            lax.fori_loop(0, CHUNK, orow, 0)
        for s in range(num_slices):
            g_hbm = ins[2 + s] if gather else None
            _zero_acc(acc, zero_v, sid, ROWS_PER_TILE, ZCH)
            plsc.subcore_barrier()
            _edge_scatter_pass(g_hbm, src2d, dst2d, src_v, dst_v, rows_v,
                               acc, sem, wid)
            plsc.subcore_barrier()
            pltpu.sync_copy(
                acc.at[pl.ds(sid * ROWS_PER_TILE, ROWS_PER_TILE)],
                outs[s].at[cid].at[pl.ds(sid * ROWS_PER_TILE, ROWS_PER_TILE)])
            if s < num_slices - 1:
                plsc.subcore_barrier()

    out_type = [jax.ShapeDtypeStruct((2, NACC, L), _f32)] * num_slices
    scratch = [
        pltpu.VMEM((KSUB, 128), jnp.int32),
        pltpu.VMEM((KSUB, 128), jnp.int32),
        pltpu.VMEM((CHUNK, L), _f32),
        pltpu.VMEM((ZCH, L), _f32),
        pltpu.VMEM_SHARED((NACC, L), _f32),
        pltpu.SemaphoreType.DMA,
    ]
    return pl.kernel(
        body, out_type=out_type, mesh=_mesh(), scratch_types=scratch,
        compiler_params=pltpu.CompilerParams(use_tc_tiling_on_sc=False))


def _pool_body(h_hbm, b2d_hbm, out_hbm, bat_v, rows_v, zero_v, acc, sem):
    del sem
    cid = lax.axis_index("c")
    sid = lax.axis_index("s")
    wid = _wid(cid, sid)
    _fill_zero(zero_v, GACC // 16)
    pltpu.sync_copy(zero_v, acc.at[pl.ds(sid * (GACC // 16), GACC // 16)])
    plsc.subcore_barrier()
    base = wid * (NPOOL // NUM_TILES)
    brow0 = wid * (NPOOL // NUM_TILES // 128)

    def pchunk(c, _):
        pltpu.sync_copy(h_hbm.at[pl.ds(base + c * POOL_CHUNK, POOL_CHUNK)], rows_v)
        pltpu.sync_copy(b2d_hbm.at[pl.ds(brow0 + c * (POOL_CHUNK // 128),
                                         POOL_CHUNK // 128)], bat_v)
        def sfire(j, _):
            pltpu.sync_copy(rows_v.at[pl.ds(j * 128, 128)],
                            acc.at[bat_v.at[j]], add=True)
            return 0
        lax.fori_loop(0, POOL_CHUNK // 128, sfire, 0)
        return 0
    lax.fori_loop(0, NPOOL // NUM_TILES // POOL_CHUNK, pchunk, 0)
    plsc.subcore_barrier()
    pltpu.sync_copy(acc.at[pl.ds(sid * (G // 16), G // 16)],
                    out_hbm.at[cid].at[pl.ds(sid * (G // 16), G // 16)])


@functools.lru_cache(maxsize=None)
def _pool_pass():
    return pl.kernel(
        _pool_body,
        out_type=jax.ShapeDtypeStruct((2, G, L), _f32),
        mesh=_mesh(),
        scratch_types=[
            pltpu.VMEM((POOL_CHUNK // 128, 128), jnp.int32),
            pltpu.VMEM((POOL_CHUNK, L), _f32),
            pltpu.VMEM((GACC // 16, L), _f32),
            pltpu.VMEM_SHARED((GACC, L), _f32),
            pltpu.SemaphoreType.DMA,
        ],
        compiler_params=pltpu.CompilerParams(use_tc_tiling_on_sc=False))


_BLK = 1000
_NBLK = N // _BLK


def _full(shape):
    return pl.BlockSpec(shape, lambda i: (0,) * len(shape))


def _rows(width):
    return pl.BlockSpec((_BLK, width), lambda i: (i, 0))


def _part():
    return pl.BlockSpec((2, _BLK, L), lambda i: (0, i, 0))


def _t0_body(x_ref, pos_ref, dp_ref, w1_ref, dinv_ref, g1_ref):
    deg = dp_ref[0, :, 0:1] + dp_ref[1, :, 0:1] + 1.0
    dinv = lax.rsqrt(deg)
    w = w1_ref[...]
    h = (x_ref[...] * w[0:1, :] + pos_ref[:, 0:1] * w[1:2, :]
         + pos_ref[:, 1:2] * w[2:3, :])
    dinv_ref[...] = dinv
    g1_ref[...] = dinv * h


def _tc0(x, pos, degp, w1):
    return pl.pallas_call(
        _t0_body,
        grid=(_NBLK,),
        in_specs=[_rows(1), _rows(2), _part(), _full((3, L))],
        out_specs=[_rows(1), _rows(L)],
        out_shape=[jax.ShapeDtypeStruct((N, 1), _f32),
                   jax.ShapeDtypeStruct((N, L), _f32)],
    )(x, pos, degp, w1)


def _mid_body(nin, nout, refs):
    ns_in, ns_out = nin // L, nout // L
    parts = refs[:ns_in]
    gs = refs[ns_in:2 * ns_in]
    dinv_ref, w_ref, b_ref = refs[2 * ns_in:2 * ns_in + 3]
    outs = refs[2 * ns_in + 3:]
    dinv = dinv_ref[...]
    b = b_ref[...]
    cols = []
    for s in range(ns_in):
        p = parts[s]
        cols.append(dinv * (p[0] + p[1] + gs[s][...]) + b[:, s * L:(s + 1) * L])
    z = jnp.maximum(jnp.concatenate(cols, axis=1) if ns_in > 1 else cols[0], 0.0)
    r = jnp.dot(z, w_ref[...], preferred_element_type=_f32)
    for t in range(ns_out):
        outs[t][...] = dinv * r[:, t * L:(t + 1) * L]


def _tc_mid(parts, gs, dinv, w, b):
    ns_in = len(gs)
    nout = w.shape[1]
    ns_out = nout // L
    body = functools.partial(_mid_body, ns_in * L, nout)

    def wrapped(*refs):
        body(refs)
    return pl.pallas_call(
        wrapped,
        grid=(_NBLK,),
        in_specs=([_part()] * ns_in + [_rows(L)] * ns_in
                  + [_rows(1), _full(w.shape), _full((1, ns_in * L))]),
        out_specs=[_rows(L)] * ns_out,
        out_shape=[jax.ShapeDtypeStruct((N, L), _f32)] * ns_out,
    )(*parts, *gs, dinv, w, b)


def _t4_body(p_ref, g_ref, dinv_ref, b_ref, out_ref):
    h = dinv_ref[...] * (p_ref[0] + p_ref[1] + g_ref[...]) + b_ref[...]
    col = lax.broadcasted_iota(jnp.int32, (_BLK, L), 1)
    out_ref[...] = jnp.where(col == 10, h + 1.0, h)


def _tc4(p4, g4, dinv, b4p):
    return pl.pallas_call(
        _t4_body,
        grid=(_NBLK,),
        in_specs=[_part(), _rows(L), _rows(1), _full((1, L))],
        out_specs=_rows(L),
        out_shape=jax.ShapeDtypeStruct((NPOOL, L), _f32),
    )(p4, g4, dinv, b4p)


_edge_pass = functools.lru_cache(maxsize=None)(_make_edge_pass)


def kernel(x, pos, edge_index, batch, W1, b1, W2, b2, W3, b3, W4, b4):
    npad = EPAD - E
    src = jnp.concatenate(
        [edge_index[0], (jnp.arange(npad, dtype=jnp.int32) * 2003) % N])
    dst = jnp.concatenate(
        [edge_index[1], N + (jnp.arange(npad, dtype=jnp.int32) % (NACC - N))])
    src2d = src.reshape(EPAD // 128, 128)
    dst2d = dst.reshape(EPAD // 128, 128)
    bpad = jnp.concatenate(
        [batch, G + (jnp.arange(NPOOL - N, dtype=jnp.int32) % (GACC - G))])
    b2d = bpad.reshape(NPOOL // 128, 128)

    (degp,) = _edge_pass(1, False)(dst2d)
    dinv, g1 = _tc0(x, pos, degp, W1)
    (p1,) = _edge_pass(1, True)(dst2d, src2d, g1)
    g2 = _tc_mid([p1], [g1], dinv, W2, b1.reshape(1, -1))
    p2 = _edge_pass(2, True)(dst2d, src2d, g2[0], g2[1])
    g3 = _tc_mid(list(p2), list(g2), dinv, W3, b2.reshape(1, -1))
    p3 = _edge_pass(4, True)(dst2d, src2d, g3[0], g3[1], g3[2], g3[3])
    W4p = jnp.pad(W4, ((0, 0), (0, L - W4.shape[1])))
    (g4,) = _tc_mid(list(p3), list(g3), dinv, W4p, b3.reshape(1, -1))
    (p4,) = _edge_pass(1, True)(dst2d, src2d, g4)
    b4p = jnp.pad(b4, (0, L - b4.shape[0])).reshape(1, L)
    h4p = _tc4(p4, g4, dinv, b4p)
    pool = _pool_pass()(h4p, b2d)
    tot = pool[0] + pool[1]
    return tot[:, :10] / jnp.maximum(tot[:, 10:11], 1.0)


# speedup vs baseline: 27.5206x; 1.5714x over previous
import functools

import jax
import jax.numpy as jnp
from jax import lax
from jax.experimental import pallas as pl
from jax.experimental.pallas import tpu as pltpu
from jax.experimental.pallas import tpu_sc as plsc

N = 100000
E = 1600000
G = 256
L = 16

NUM_TILES = 32
CHUNK = 1024
KSUB = CHUNK // 128
EPAD = 1605632
TILE_CHUNKS = EPAD // NUM_TILES // CHUNK
EROWS_PER_TILE = EPAD // NUM_TILES // 128

NACC = 102400
ROWS_PER_TILE = NACC // 16
ZCH = 320

NPOOL = 102400
POOL_CHUNK = 640
GACC = G + 32

_f32 = jnp.float32


@functools.lru_cache(maxsize=None)
def _mesh():
    return plsc.VectorSubcoreMesh(core_axis_name="c", subcore_axis_name="s")


def _wid(cid, sid):
    return sid * 2 + cid


def _fill_zero(zref, nrows):
    def zrow(i, _):
        zref[i, :] = jnp.zeros((L,), _f32)
        return 0
    lax.fori_loop(0, nrows, zrow, 0)


def _zero_acc(acc, zero_v, sid, rows_per_tile, zch):
    def zchunk(i, _):
        pltpu.sync_copy(zero_v, acc.at[pl.ds(sid * rows_per_tile + i * zch, zch)])
        return 0
    lax.fori_loop(0, rows_per_tile // zch, zchunk, 0)


def _edge_scatter_pass(g_hbm, src2d, dst2d, src_v, dst_v, rows_v, acc, sem, wid):
    def echunk(c, _):
        row0 = wid * EROWS_PER_TILE + c * KSUB
        if g_hbm is not None:
            pltpu.sync_copy(src2d.at[pl.ds(row0, KSUB)], src_v)
        pltpu.sync_copy(dst2d.at[pl.ds(row0, KSUB)], dst_v)
        if g_hbm is not None:
            def gfire(j, _):
                pltpu.async_copy(
                    g_hbm.at[src_v.at[j]], rows_v.at[pl.ds(j * 128, 128)], sem)
                return 0
            lax.fori_loop(0, KSUB, gfire, 0)
            pltpu.make_async_copy(
                g_hbm.at[pl.ds(0, CHUNK)], rows_v, sem).wait()
        def sfire(j, _):
            pltpu.sync_copy(
                rows_v.at[pl.ds(j * 128, 128)], acc.at[dst_v.at[j]], add=True)
            return 0
        lax.fori_loop(0, KSUB, sfire, 0)
        return 0
    lax.fori_loop(0, TILE_CHUNKS, echunk, 0)


def _make_edge_pass(num_slices, gather):
    n_in = (2 + num_slices) if gather else 1

    def body(*refs):
        ins = refs[:n_in]
        outs = refs[n_in:n_in + num_slices]
        src_v, dst_v, rows_v, zero_v, acc, sem = refs[n_in + num_slices:]
        dst2d = ins[0]
        src2d = ins[1] if gather else None
        cid = lax.axis_index("c")
        sid = lax.axis_index("s")
        wid = _wid(cid, sid)
        _fill_zero(zero_v, ZCH)
        if not gather:
            def orow(i, _):
                rows_v[i, :] = jnp.ones((L,), _f32)
                return 0
            lax.fori_loop(0, CHUNK, orow, 0)
        for s in range(num_slices):
            g_hbm = ins[2 + s] if gather else None
            _zero_acc(acc, zero_v, sid, ROWS_PER_TILE, ZCH)
            plsc.subcore_barrier()
            _edge_scatter_pass(g_hbm, src2d, dst2d, src_v, dst_v, rows_v,
                               acc, sem, wid)
            plsc.subcore_barrier()
            pltpu.sync_copy(
                acc.at[pl.ds(sid * ROWS_PER_TILE, ROWS_PER_TILE)],
                outs[s].at[cid].at[pl.ds(sid * ROWS_PER_TILE, ROWS_PER_TILE)])
            if s < num_slices - 1:
                plsc.subcore_barrier()

    out_type = [jax.ShapeDtypeStruct((2, NACC, L), _f32)] * num_slices
    scratch = [
        pltpu.VMEM((KSUB, 128), jnp.int32),
        pltpu.VMEM((KSUB, 128), jnp.int32),
        pltpu.VMEM((CHUNK, L), _f32),
        pltpu.VMEM((ZCH, L), _f32),
        pltpu.VMEM_SHARED((NACC, L), _f32),
        pltpu.SemaphoreType.DMA,
    ]
    return pl.kernel(
        body, out_type=out_type, mesh=_mesh(), scratch_types=scratch,
        compiler_params=pltpu.CompilerParams(use_tc_tiling_on_sc=False))


def _pool_body(h_hbm, b2d_hbm, out_hbm, bat_v, rows_v, zero_v, acc, sem):
    del sem
    cid = lax.axis_index("c")
    sid = lax.axis_index("s")
    wid = _wid(cid, sid)
    _fill_zero(zero_v, GACC // 16)
    pltpu.sync_copy(zero_v, acc.at[pl.ds(sid * (GACC // 16), GACC // 16)])
    plsc.subcore_barrier()
    base = wid * (NPOOL // NUM_TILES)
    brow0 = wid * (NPOOL // NUM_TILES // 128)

    def pchunk(c, _):
        pltpu.sync_copy(h_hbm.at[pl.ds(base + c * POOL_CHUNK, POOL_CHUNK)], rows_v)
        pltpu.sync_copy(b2d_hbm.at[pl.ds(brow0 + c * (POOL_CHUNK // 128),
                                         POOL_CHUNK // 128)], bat_v)
        def sfire(j, _):
            pltpu.sync_copy(rows_v.at[pl.ds(j * 128, 128)],
                            acc.at[bat_v.at[j]], add=True)
            return 0
        lax.fori_loop(0, POOL_CHUNK // 128, sfire, 0)
        return 0
    lax.fori_loop(0, NPOOL // NUM_TILES // POOL_CHUNK, pchunk, 0)
    plsc.subcore_barrier()
    pltpu.sync_copy(acc.at[pl.ds(sid * (G // 16), G // 16)],
                    out_hbm.at[cid].at[pl.ds(sid * (G // 16), G // 16)])


@functools.lru_cache(maxsize=None)
def _pool_pass():
    return pl.kernel(
        _pool_body,
        out_type=jax.ShapeDtypeStruct((2, G, L), _f32),
        mesh=_mesh(),
        scratch_types=[
            pltpu.VMEM((POOL_CHUNK // 128, 128), jnp.int32),
            pltpu.VMEM((POOL_CHUNK, L), _f32),
            pltpu.VMEM((GACC // 16, L), _f32),
            pltpu.VMEM_SHARED((GACC, L), _f32),
            pltpu.SemaphoreType.DMA,
        ],
        compiler_params=pltpu.CompilerParams(use_tc_tiling_on_sc=False))


_BLKW = 128
_NW = NACC // 8
_NBLK = _NW // _BLKW
_NACCW = NACC // 8


def _full(shape):
    return pl.BlockSpec(shape, lambda i: (0,) * len(shape))


def _rowsw():
    return pl.BlockSpec((_BLKW, 128), lambda i: (i, 0))


def _partw():
    return pl.BlockSpec((2, _BLKW, 128), lambda i: (0, i, 0))


def _t0_body(h0_ref, dp_ref, bd_ref, w1_ref, dinv_ref, g1_ref):
    psum = dp_ref[0] + dp_ref[1]
    deg = jnp.dot(psum, bd_ref[...], preferred_element_type=_f32) + 1.0
    dinv = lax.rsqrt(deg)
    hw = jnp.dot(h0_ref[...], w1_ref[...], preferred_element_type=_f32)
    dinv_ref[...] = dinv
    g1_ref[...] = dinv * hw


def _tc0(h0r, degp_w, bd, w1k):
    return pl.pallas_call(
        _t0_body,
        grid=(_NBLK,),
        in_specs=[pl.BlockSpec((_BLKW, 24), lambda i: (i, 0)), _partw(),
                  _full((128, 128)), _full((24, 128))],
        out_specs=[_rowsw(), _rowsw()],
        out_shape=[jax.ShapeDtypeStruct((_NW, 128), _f32),
                   jax.ShapeDtypeStruct((_NW, 128), _f32)],
    )(h0r, degp_w, bd, w1k)


def _mid_body(ns_in, ns_out, refs):
    parts = refs[:ns_in]
    gs = refs[ns_in:2 * ns_in]
    dinv_ref, w_ref, b_ref = refs[2 * ns_in:2 * ns_in + 3]
    outs = refs[2 * ns_in + 3:]
    dinv = dinv_ref[...]
    b = b_ref[...]
    cols = []
    for s in range(ns_in):
        p = parts[s]
        cols.append(dinv * (p[0] + p[1] + gs[s][...])
                    + b[:, s * 128:(s + 1) * 128])
    z = jnp.maximum(jnp.concatenate(cols, axis=1) if ns_in > 1 else cols[0],
                    0.0)
    r = jnp.dot(z, w_ref[...], preferred_element_type=_f32)
    for t in range(ns_out):
        outs[t][...] = dinv * r[:, t * 128:(t + 1) * 128]


def _tc_mid(parts, gs, dinv, wk, bt):
    ns_in = len(gs)
    ns_out = wk.shape[1] // 128
    body = functools.partial(_mid_body, ns_in, ns_out)

    def wrapped(*refs):
        body(refs)
    return pl.pallas_call(
        wrapped,
        grid=(_NBLK,),
        in_specs=([_partw()] * ns_in + [_rowsw()] * ns_in
                  + [_rowsw(), _full(wk.shape), _full((1, ns_in * 128))]),
        out_specs=[_rowsw()] * ns_out,
        out_shape=[jax.ShapeDtypeStruct((_NW, 128), _f32)] * ns_out,
    )(*parts, *gs, dinv, wk, bt)


def _t4_body(p_ref, g_ref, dinv_ref, b_ref, out_ref):
    h = dinv_ref[...] * (p_ref[0] + p_ref[1] + g_ref[...]) + b_ref[...]
    lane = lax.broadcasted_iota(jnp.int32, (_BLKW, 128), 1)
    out_ref[...] = jnp.where(lane % L == 10, h + 1.0, h)


def _tc4(p4, g4, dinv, b4t):
    return pl.pallas_call(
        _t4_body,
        grid=(_NBLK,),
        in_specs=[_partw(), _rowsw(), _rowsw(), _full((1, 128))],
        out_specs=_rowsw(),
        out_shape=jax.ShapeDtypeStruct((NPOOL // 8, 128), _f32),
    )(p4, g4, dinv, b4t)


def _kron8(wblk):
    return jnp.kron(jnp.eye(8, dtype=_f32), wblk)


def _big_w(w):
    si, so = w.shape[0] // L, w.shape[1] // L
    return jnp.concatenate([
        jnp.concatenate([_kron8(w[s * L:(s + 1) * L, t * L:(t + 1) * L])
                         for t in range(so)], axis=1)
        for s in range(si)], axis=0)


def _bias_tile(b):
    return jnp.concatenate(
        [jnp.tile(b[s * L:(s + 1) * L], 8) for s in range(b.shape[0] // L)]
    ).reshape(1, -1)


_edge_pass = functools.lru_cache(maxsize=None)(_make_edge_pass)


def kernel(x, pos, edge_index, batch, W1, b1, W2, b2, W3, b3, W4, b4):
    npad = EPAD - E
    src = jnp.concatenate(
        [edge_index[0], (jnp.arange(npad, dtype=jnp.int32) * 2003) % N])
    dst = jnp.concatenate(
        [edge_index[1], N + (jnp.arange(npad, dtype=jnp.int32) % (NACC - N))])
    src2d = src.reshape(EPAD // 128, 128)
    dst2d = dst.reshape(EPAD // 128, 128)
    bpad = jnp.concatenate(
        [batch, G + (jnp.arange(NPOOL - N, dtype=jnp.int32) % (GACC - G))])
    b2d = bpad.reshape(NPOOL // 128, 128)

    def wide(p):
        return p.reshape(2, _NACCW, 128)

    def tbl(gw):
        return gw.reshape(NACC, L)

    h0r = jnp.pad(jnp.concatenate([x, pos], axis=1).reshape(N // 8, 24),
                  ((0, _NW - N // 8), (0, 0)))
    ones_row = jnp.zeros((L, L), _f32).at[0, :].set(1.0)
    bd = _kron8(ones_row)
    w1k = _kron8(W1)

    (degp,) = _edge_pass(1, False)(dst2d)
    dinv, g1 = _tc0(h0r, wide(degp), bd, w1k)
    (p1,) = _edge_pass(1, True)(dst2d, src2d, tbl(g1))
    g2 = _tc_mid([wide(p1)], [g1], dinv, _big_w(W2), _bias_tile(b1))
    p2 = _edge_pass(2, True)(dst2d, src2d, tbl(g2[0]), tbl(g2[1]))
    g3 = _tc_mid([wide(p) for p in p2], list(g2), dinv, _big_w(W3),
                 _bias_tile(b2))
    p3 = _edge_pass(4, True)(dst2d, src2d, tbl(g3[0]), tbl(g3[1]),
                             tbl(g3[2]), tbl(g3[3]))
    W4p = jnp.pad(W4, ((0, 0), (0, L - W4.shape[1])))
    (g4,) = _tc_mid([wide(p) for p in p3], list(g3), dinv, _big_w(W4p),
                    _bias_tile(b3))
    (p4,) = _edge_pass(1, True)(dst2d, src2d, tbl(g4))
    b4t = _bias_tile(jnp.pad(b4, (0, L - b4.shape[0])))
    h4p = _tc4(wide(p4), g4, dinv, b4t)
    pool = _pool_pass()(h4p.reshape(NPOOL, L), b2d)
    tot = pool[0] + pool[1]
    return tot[:, :10] / jnp.maximum(tot[:, 10:11], 1.0)


# speedup vs baseline: 34.3917x; 1.2497x over previous
import functools

import jax
import jax.numpy as jnp
from jax import lax
from jax.experimental import pallas as pl
from jax.experimental.pallas import tpu as pltpu
from jax.experimental.pallas import tpu_sc as plsc

N = 100000
E = 1600000
G = 256
L = 16

NUM_TILES = 32
CHUNK = 512
KSUB = CHUNK // 128
EPAD = 1605632
TILE_CHUNKS = EPAD // NUM_TILES // CHUNK
NPAIR = TILE_CHUNKS // 2
EROWS_PER_TILE = EPAD // NUM_TILES // 128

NACC = 102400
ROWS_PER_TILE = NACC // 16
ZCH = 320

NPOOL = 102400
POOL_CHUNK = 640
GACC = G + 32

_f32 = jnp.float32


@functools.lru_cache(maxsize=None)
def _mesh():
    return plsc.VectorSubcoreMesh(core_axis_name="c", subcore_axis_name="s")


def _wid(cid, sid):
    return sid * 2 + cid


def _fill_zero(zref, nrows):
    def zrow(i, _):
        zref[i, :] = jnp.zeros((L,), _f32)
        return 0
    lax.fori_loop(0, nrows, zrow, 0)


def _zero_acc(acc, zero_v, sid, rows_per_tile, zch):
    def zchunk(i, _):
        pltpu.sync_copy(zero_v, acc.at[pl.ds(sid * rows_per_tile + i * zch, zch)])
        return 0
    lax.fori_loop(0, rows_per_tile // zch, zchunk, 0)


def _edge_scatter_pass(g_hbm, src2d, dst2d, src_v, dst_v, rows_v, acc,
                       sem_g, sem_s, wid):
    base = wid * EROWS_PER_TILE

    def load_idx(c, x):
        if g_hbm is not None:
            pltpu.sync_copy(src2d.at[pl.ds(base + c * KSUB, KSUB)], src_v[x])
        pltpu.sync_copy(dst2d.at[pl.ds(base + c * KSUB, KSUB)], dst_v[x])

    def fire_gather(x):
        for j in range(KSUB):
            pltpu.async_copy(g_hbm.at[src_v[x].at[j]],
                             rows_v[x].at[pl.ds(j * 128, 128)], sem_g[x])

    def drain_gather(x):
        pltpu.make_async_copy(g_hbm.at[pl.ds(0, CHUNK)], rows_v[x],
                              sem_g[x]).wait()

    def fire_scatter(x):
        r = rows_v[x] if g_hbm is not None else rows_v[0]
        for j in range(KSUB):
            pltpu.async_copy(r.at[pl.ds(j * 128, 128)],
                             acc.at[dst_v[x].at[j]], sem_s[x], add=True)

    def drain_scatter(x):
        pltpu.make_async_copy(rows_v[x] if g_hbm is not None else rows_v[0],
                              acc.at[pl.ds(0, CHUNK)], sem_s[x]).wait()

    if g_hbm is None:
        def dpair(p, _):
            for x in range(2):
                @pl.when(p > 0)
                def _():
                    drain_scatter(x)
                load_idx(2 * p + x, x)
                fire_scatter(x)
            return 0
        lax.fori_loop(0, NPAIR, dpair, 0)
        drain_scatter(0)
        drain_scatter(1)
        return

    load_idx(0, 0)
    fire_gather(0)

    def pipe(p, _):
        @pl.when(p > 0)
        def _():
            drain_scatter(1)
        load_idx(2 * p + 1, 1)
        fire_gather(1)
        drain_gather(0)
        fire_scatter(0)

        @pl.when(p < NPAIR - 1)
        def _():
            drain_scatter(0)
            load_idx(2 * p + 2, 0)
            fire_gather(0)
        drain_gather(1)
        fire_scatter(1)
        return 0
    lax.fori_loop(0, NPAIR, pipe, 0)
    drain_scatter(0)
    drain_scatter(1)


def _make_edge_pass(num_slices, gather):
    n_in = (2 + num_slices) if gather else 1

    def body(*refs):
        ins = refs[:n_in]
        outs = refs[n_in:n_in + num_slices]
        (s0, s1, d0, d1, r0, r1, zero_v, acc,
         sg0, sg1, ss0, ss1) = refs[n_in + num_slices:]
        src_v, dst_v, rows_v = (s0, s1), (d0, d1), (r0, r1)
        sem_g, sem_s = (sg0, sg1), (ss0, ss1)
        dst2d = ins[0]
        src2d = ins[1] if gather else None
        cid = lax.axis_index("c")
        sid = lax.axis_index("s")
        wid = _wid(cid, sid)
        _fill_zero(zero_v, ZCH)
        if not gather:
            def orow(i, _):
                r0[i, :] = jnp.ones((L,), _f32)
                return 0
            lax.fori_loop(0, CHUNK, orow, 0)
        for s in range(num_slices):
            g_hbm = ins[2 + s] if gather else None
            _zero_acc(acc, zero_v, sid, ROWS_PER_TILE, ZCH)
            plsc.subcore_barrier()
            _edge_scatter_pass(g_hbm, src2d, dst2d, src_v, dst_v, rows_v,
                               acc, sem_g, sem_s, wid)
            plsc.subcore_barrier()
            pltpu.sync_copy(
                acc.at[pl.ds(sid * ROWS_PER_TILE, ROWS_PER_TILE)],
                outs[s].at[cid].at[pl.ds(sid * ROWS_PER_TILE, ROWS_PER_TILE)])
            if s < num_slices - 1:
                plsc.subcore_barrier()

    out_type = [jax.ShapeDtypeStruct((2, NACC, L), _f32)] * num_slices
    scratch = [
        pltpu.VMEM((KSUB, 128), jnp.int32),
        pltpu.VMEM((KSUB, 128), jnp.int32),
        pltpu.VMEM((KSUB, 128), jnp.int32),
        pltpu.VMEM((KSUB, 128), jnp.int32),
        pltpu.VMEM((CHUNK, L), _f32),
        pltpu.VMEM((CHUNK, L), _f32),
        pltpu.VMEM((ZCH, L), _f32),
        pltpu.VMEM_SHARED((NACC, L), _f32),
        pltpu.SemaphoreType.DMA,
        pltpu.SemaphoreType.DMA,
        pltpu.SemaphoreType.DMA,
        pltpu.SemaphoreType.DMA,
    ]
    return pl.kernel(
        body, out_type=out_type, mesh=_mesh(), scratch_types=scratch,
        compiler_params=pltpu.CompilerParams(use_tc_tiling_on_sc=False))


def _pool_body(h_hbm, b2d_hbm, out_hbm, bat_v, rows_v, zero_v, acc, sem):
    del sem
    cid = lax.axis_index("c")
    sid = lax.axis_index("s")
    wid = _wid(cid, sid)
    _fill_zero(zero_v, GACC // 16)
    pltpu.sync_copy(zero_v, acc.at[pl.ds(sid * (GACC // 16), GACC // 16)])
    plsc.subcore_barrier()
    base = wid * (NPOOL // NUM_TILES)
    brow0 = wid * (NPOOL // NUM_TILES // 128)

    def pchunk(c, _):
        pltpu.sync_copy(h_hbm.at[pl.ds(base + c * POOL_CHUNK, POOL_CHUNK)], rows_v)
        pltpu.sync_copy(b2d_hbm.at[pl.ds(brow0 + c * (POOL_CHUNK // 128),
                                         POOL_CHUNK // 128)], bat_v)
        def sfire(j, _):
            pltpu.sync_copy(rows_v.at[pl.ds(j * 128, 128)],
                            acc.at[bat_v.at[j]], add=True)
            return 0
        lax.fori_loop(0, POOL_CHUNK // 128, sfire, 0)
        return 0
    lax.fori_loop(0, NPOOL // NUM_TILES // POOL_CHUNK, pchunk, 0)
    plsc.subcore_barrier()
    pltpu.sync_copy(acc.at[pl.ds(sid * (G // 16), G // 16)],
                    out_hbm.at[cid].at[pl.ds(sid * (G // 16), G // 16)])


@functools.lru_cache(maxsize=None)
def _pool_pass():
    return pl.kernel(
        _pool_body,
        out_type=jax.ShapeDtypeStruct((2, G, L), _f32),
        mesh=_mesh(),
        scratch_types=[
            pltpu.VMEM((POOL_CHUNK // 128, 128), jnp.int32),
            pltpu.VMEM((POOL_CHUNK, L), _f32),
            pltpu.VMEM((GACC // 16, L), _f32),
            pltpu.VMEM_SHARED((GACC, L), _f32),
            pltpu.SemaphoreType.DMA,
        ],
        compiler_params=pltpu.CompilerParams(use_tc_tiling_on_sc=False))


_BLKW = 128
_NW = NACC // 8
_NBLK = _NW // _BLKW
_NACCW = NACC // 8


def _full(shape):
    return pl.BlockSpec(shape, lambda i: (0,) * len(shape))


def _rowsw():
    return pl.BlockSpec((_BLKW, 128), lambda i: (i, 0))


def _partw():
    return pl.BlockSpec((2, _BLKW, 128), lambda i: (0, i, 0))


def _t0_body(h0_ref, dp_ref, bd_ref, w1_ref, dinv_ref, g1_ref):
    psum = dp_ref[0] + dp_ref[1]
    deg = jnp.dot(psum, bd_ref[...], preferred_element_type=_f32) + 1.0
    dinv = lax.rsqrt(deg)
    hw = jnp.dot(h0_ref[...], w1_ref[...], preferred_element_type=_f32)
    dinv_ref[...] = dinv
    g1_ref[...] = dinv * hw


def _tc0(h0r, degp_w, bd, w1k):
    return pl.pallas_call(
        _t0_body,
        grid=(_NBLK,),
        in_specs=[pl.BlockSpec((_BLKW, 24), lambda i: (i, 0)), _partw(),
                  _full((128, 128)), _full((24, 128))],
        out_specs=[_rowsw(), _rowsw()],
        out_shape=[jax.ShapeDtypeStruct((_NW, 128), _f32),
                   jax.ShapeDtypeStruct((_NW, 128), _f32)],
    )(h0r, degp_w, bd, w1k)


def _mid_body(ns_in, ns_out, refs):
    parts = refs[:ns_in]
    gs = refs[ns_in:2 * ns_in]
    dinv_ref, w_ref, b_ref = refs[2 * ns_in:2 * ns_in + 3]
    outs = refs[2 * ns_in + 3:]
    dinv = dinv_ref[...]
    b = b_ref[...]
    cols = []
    for s in range(ns_in):
        p = parts[s]
        cols.append(dinv * (p[0] + p[1] + gs[s][...])
                    + b[:, s * 128:(s + 1) * 128])
    z = jnp.maximum(jnp.concatenate(cols, axis=1) if ns_in > 1 else cols[0],
                    0.0)
    r = jnp.dot(z, w_ref[...], preferred_element_type=_f32)
    for t in range(ns_out):
        outs[t][...] = dinv * r[:, t * 128:(t + 1) * 128]


def _tc_mid(parts, gs, dinv, wk, bt):
    ns_in = len(gs)
    ns_out = wk.shape[1] // 128
    body = functools.partial(_mid_body, ns_in, ns_out)

    def wrapped(*refs):
        body(refs)
    return pl.pallas_call(
        wrapped,
        grid=(_NBLK,),
        in_specs=([_partw()] * ns_in + [_rowsw()] * ns_in
                  + [_rowsw(), _full(wk.shape), _full((1, ns_in * 128))]),
        out_specs=[_rowsw()] * ns_out,
        out_shape=[jax.ShapeDtypeStruct((_NW, 128), _f32)] * ns_out,
    )(*parts, *gs, dinv, wk, bt)


def _t4_body(p_ref, g_ref, dinv_ref, b_ref, out_ref):
    h = dinv_ref[...] * (p_ref[0] + p_ref[1] + g_ref[...]) + b_ref[...]
    lane = lax.broadcasted_iota(jnp.int32, (_BLKW, 128), 1)
    out_ref[...] = jnp.where(lane % L == 10, h + 1.0, h)


def _tc4(p4, g4, dinv, b4t):
    return pl.pallas_call(
        _t4_body,
        grid=(_NBLK,),
        in_specs=[_partw(), _rowsw(), _rowsw(), _full((1, 128))],
        out_specs=_rowsw(),
        out_shape=jax.ShapeDtypeStruct((NPOOL // 8, 128), _f32),
    )(p4, g4, dinv, b4t)


def _kron8(wblk):
    return jnp.kron(jnp.eye(8, dtype=_f32), wblk)


def _big_w(w):
    si, so = w.shape[0] // L, w.shape[1] // L
    return jnp.concatenate([
        jnp.concatenate([_kron8(w[s * L:(s + 1) * L, t * L:(t + 1) * L])
                         for t in range(so)], axis=1)
        for s in range(si)], axis=0)


def _bias_tile(b):
    return jnp.concatenate(
        [jnp.tile(b[s * L:(s + 1) * L], 8) for s in range(b.shape[0] // L)]
    ).reshape(1, -1)


_edge_pass = functools.lru_cache(maxsize=None)(_make_edge_pass)


def kernel(x, pos, edge_index, batch, W1, b1, W2, b2, W3, b3, W4, b4):
    npad = EPAD - E
    src = jnp.concatenate(
        [edge_index[0], (jnp.arange(npad, dtype=jnp.int32) * 2003) % N])
    dst = jnp.concatenate(
        [edge_index[1], N + (jnp.arange(npad, dtype=jnp.int32) % (NACC - N))])
    src2d = src.reshape(EPAD // 128, 128)
    dst2d = dst.reshape(EPAD // 128, 128)
    bpad = jnp.concatenate(
        [batch, G + (jnp.arange(NPOOL - N, dtype=jnp.int32) % (GACC - G))])
    b2d = bpad.reshape(NPOOL // 128, 128)

    def wide(p):
        return p.reshape(2, _NACCW, 128)

    def tbl(gw):
        return gw.reshape(NACC, L)

    h0r = jnp.pad(jnp.concatenate([x, pos], axis=1).reshape(N // 8, 24),
                  ((0, _NW - N // 8), (0, 0)))
    ones_row = jnp.zeros((L, L), _f32).at[0, :].set(1.0)
    bd = _kron8(ones_row)
    w1k = _kron8(W1)

    (degp,) = _edge_pass(1, False)(dst2d)
    dinv, g1 = _tc0(h0r, wide(degp), bd, w1k)
    (p1,) = _edge_pass(1, True)(dst2d, src2d, tbl(g1))
    g2 = _tc_mid([wide(p1)], [g1], dinv, _big_w(W2), _bias_tile(b1))
    p2 = _edge_pass(2, True)(dst2d, src2d, tbl(g2[0]), tbl(g2[1]))
    g3 = _tc_mid([wide(p) for p in p2], list(g2), dinv, _big_w(W3),
                 _bias_tile(b2))
    p3 = _edge_pass(4, True)(dst2d, src2d, tbl(g3[0]), tbl(g3[1]),
                             tbl(g3[2]), tbl(g3[3]))
    W4p = jnp.pad(W4, ((0, 0), (0, L - W4.shape[1])))
    (g4,) = _tc_mid([wide(p) for p in p3], list(g3), dinv, _big_w(W4p),
                    _bias_tile(b3))
    (p4,) = _edge_pass(1, True)(dst2d, src2d, tbl(g4))
    b4t = _bias_tile(jnp.pad(b4, (0, L - b4.shape[0])))
    h4p = _tc4(wide(p4), g4, dinv, b4t)
    pool = _pool_pass()(h4p.reshape(NPOOL, L), b2d)
    tot = pool[0] + pool[1]
    return tot[:, :10] / jnp.maximum(tot[:, 10:11], 1.0)


# speedup vs baseline: 34.4048x; 1.0004x over previous
import functools

import jax
import jax.numpy as jnp
from jax import lax
from jax.experimental import pallas as pl
from jax.experimental.pallas import tpu as pltpu
from jax.experimental.pallas import tpu_sc as plsc

N = 100000
E = 1600000
G = 256
L = 16

NUM_TILES = 32
CHUNK = 512
KSUB = CHUNK // 128
EPAD = 1605632
TILE_CHUNKS = EPAD // NUM_TILES // CHUNK
NPAIR = TILE_CHUNKS // 2
EROWS_PER_TILE = EPAD // NUM_TILES // 128
EROWS_MAIN = E // 128

NACC = 102400
ROWS_PER_TILE = NACC // 16
ZCH = 320

NPOOL = 102400
POOL_CHUNK = 640
GACC = G + 32

_f32 = jnp.float32


@functools.lru_cache(maxsize=None)
def _mesh():
    return plsc.VectorSubcoreMesh(core_axis_name="c", subcore_axis_name="s")


def _wid(cid, sid):
    return sid * 2 + cid


def _fill_zero(zref, nrows):
    def zrow(i, _):
        zref[i, :] = jnp.zeros((L,), _f32)
        return 0
    lax.fori_loop(0, nrows, zrow, 0)


def _zero_acc(acc, zero_v, sid, rows_per_tile, zch):
    def zchunk(i, _):
        pltpu.sync_copy(zero_v, acc.at[pl.ds(sid * rows_per_tile + i * zch, zch)])
        return 0
    lax.fori_loop(0, rows_per_tile // zch, zchunk, 0)


def _edge_scatter_pass(g_hbm, src2d, dst2d, src_v, dst_v, rows_v, acc,
                       sem_g, sem_s, wid):
    base = wid * EROWS_PER_TILE
    src_m, src_p = src2d
    dst_m, dst_p = dst2d

    def load_idx(c, x):
        row0 = base + c * KSUB

        @pl.when(row0 < EROWS_MAIN)
        def _():
            if g_hbm is not None:
                pltpu.sync_copy(src_m.at[pl.ds(row0, KSUB)], src_v[x])
            pltpu.sync_copy(dst_m.at[pl.ds(row0, KSUB)], dst_v[x])

        @pl.when(row0 >= EROWS_MAIN)
        def _():
            if g_hbm is not None:
                pltpu.sync_copy(src_p.at[pl.ds(row0 - EROWS_MAIN, KSUB)],
                                src_v[x])
            pltpu.sync_copy(dst_p.at[pl.ds(row0 - EROWS_MAIN, KSUB)],
                            dst_v[x])

    def fire_gather(x):
        for j in range(KSUB):
            pltpu.async_copy(g_hbm.at[src_v[x].at[j]],
                             rows_v[x].at[pl.ds(j * 128, 128)], sem_g[x])

    def drain_gather(x):
        pltpu.make_async_copy(g_hbm.at[pl.ds(0, CHUNK)], rows_v[x],
                              sem_g[x]).wait()

    def fire_scatter(x):
        r = rows_v[x] if g_hbm is not None else rows_v[0]
        for j in range(KSUB):
            pltpu.async_copy(r.at[pl.ds(j * 128, 128)],
                             acc.at[dst_v[x].at[j]], sem_s[x], add=True)

    def drain_scatter(x):
        pltpu.make_async_copy(rows_v[x] if g_hbm is not None else rows_v[0],
                              acc.at[pl.ds(0, CHUNK)], sem_s[x]).wait()

    if g_hbm is None:
        def dpair(p, _):
            for x in range(2):
                @pl.when(p > 0)
                def _():
                    drain_scatter(x)
                load_idx(2 * p + x, x)
                fire_scatter(x)
            return 0
        lax.fori_loop(0, NPAIR, dpair, 0)
        drain_scatter(0)
        drain_scatter(1)
        return

    load_idx(0, 0)
    fire_gather(0)

    def pipe(p, _):
        @pl.when(p > 0)
        def _():
            drain_scatter(1)
        load_idx(2 * p + 1, 1)
        fire_gather(1)
        drain_gather(0)
        fire_scatter(0)

        @pl.when(p < NPAIR - 1)
        def _():
            drain_scatter(0)
            load_idx(2 * p + 2, 0)
            fire_gather(0)
        drain_gather(1)
        fire_scatter(1)
        return 0
    lax.fori_loop(0, NPAIR, pipe, 0)
    drain_scatter(0)
    drain_scatter(1)


def _make_edge_pass(num_slices, gather):
    n_in = (4 + num_slices) if gather else 2

    def body(*refs):
        ins = refs[:n_in]
        outs = refs[n_in:n_in + num_slices]
        (s0, s1, d0, d1, r0, r1, zero_v, acc,
         sg0, sg1, ss0, ss1) = refs[n_in + num_slices:]
        src_v, dst_v, rows_v = (s0, s1), (d0, d1), (r0, r1)
        sem_g, sem_s = (sg0, sg1), (ss0, ss1)
        dst2d = (ins[0], ins[1])
        src2d = (ins[2], ins[3]) if gather else (None, None)
        cid = lax.axis_index("c")
        sid = lax.axis_index("s")
        wid = _wid(cid, sid)
        _fill_zero(zero_v, ZCH)
        if not gather:
            def orow(i, _):
                r0[i, :] = jnp.ones((L,), _f32)
                return 0
            lax.fori_loop(0, CHUNK, orow, 0)
        for s in range(num_slices):
            g_hbm = ins[4 + s] if gather else None
            _zero_acc(acc, zero_v, sid, ROWS_PER_TILE, ZCH)
            plsc.subcore_barrier()
            _edge_scatter_pass(g_hbm, src2d, dst2d, src_v, dst_v, rows_v,
                               acc, sem_g, sem_s, wid)
            plsc.subcore_barrier()
            pltpu.sync_copy(
                acc.at[pl.ds(sid * ROWS_PER_TILE, ROWS_PER_TILE)],
                outs[s].at[cid].at[pl.ds(sid * ROWS_PER_TILE, ROWS_PER_TILE)])
            if s < num_slices - 1:
                plsc.subcore_barrier()

    out_type = [jax.ShapeDtypeStruct((2, NACC, L), _f32)] * num_slices
    scratch = [
        pltpu.VMEM((KSUB, 128), jnp.int32),
        pltpu.VMEM((KSUB, 128), jnp.int32),
        pltpu.VMEM((KSUB, 128), jnp.int32),
        pltpu.VMEM((KSUB, 128), jnp.int32),
        pltpu.VMEM((CHUNK, L), _f32),
        pltpu.VMEM((CHUNK, L), _f32),
        pltpu.VMEM((ZCH, L), _f32),
        pltpu.VMEM_SHARED((NACC, L), _f32),
        pltpu.SemaphoreType.DMA,
        pltpu.SemaphoreType.DMA,
        pltpu.SemaphoreType.DMA,
        pltpu.SemaphoreType.DMA,
    ]
    return pl.kernel(
        body, out_type=out_type, mesh=_mesh(), scratch_types=scratch,
        compiler_params=pltpu.CompilerParams(use_tc_tiling_on_sc=False))


def _pool_body(h_hbm, b2d_hbm, out_hbm, bat_v, rows_v, zero_v, acc, sem):
    del sem
    cid = lax.axis_index("c")
    sid = lax.axis_index("s")
    wid = _wid(cid, sid)
    _fill_zero(zero_v, GACC // 16)
    pltpu.sync_copy(zero_v, acc.at[pl.ds(sid * (GACC // 16), GACC // 16)])
    plsc.subcore_barrier()
    base = wid * (NPOOL // NUM_TILES)
    brow0 = wid * (NPOOL // NUM_TILES // 128)

    def pchunk(c, _):
        pltpu.sync_copy(h_hbm.at[pl.ds(base + c * POOL_CHUNK, POOL_CHUNK)], rows_v)
        pltpu.sync_copy(b2d_hbm.at[pl.ds(brow0 + c * (POOL_CHUNK // 128),
                                         POOL_CHUNK // 128)], bat_v)
        def sfire(j, _):
            pltpu.sync_copy(rows_v.at[pl.ds(j * 128, 128)],
                            acc.at[bat_v.at[j]], add=True)
            return 0
        lax.fori_loop(0, POOL_CHUNK // 128, sfire, 0)
        return 0
    lax.fori_loop(0, NPOOL // NUM_TILES // POOL_CHUNK, pchunk, 0)
    plsc.subcore_barrier()
    pltpu.sync_copy(acc.at[pl.ds(sid * (G // 16), G // 16)],
                    out_hbm.at[cid].at[pl.ds(sid * (G // 16), G // 16)])


@functools.lru_cache(maxsize=None)
def _pool_pass():
    return pl.kernel(
        _pool_body,
        out_type=jax.ShapeDtypeStruct((2, G, L), _f32),
        mesh=_mesh(),
        scratch_types=[
            pltpu.VMEM((POOL_CHUNK // 128, 128), jnp.int32),
            pltpu.VMEM((POOL_CHUNK, L), _f32),
            pltpu.VMEM((GACC // 16, L), _f32),
            pltpu.VMEM_SHARED((GACC, L), _f32),
            pltpu.SemaphoreType.DMA,
        ],
        compiler_params=pltpu.CompilerParams(use_tc_tiling_on_sc=False))


_BLKW = 128
_NW = NACC // 8
_NBLK = _NW // _BLKW
_NACCW = NACC // 8


def _full(shape):
    return pl.BlockSpec(shape, lambda i: (0,) * len(shape))


def _rowsw():
    return pl.BlockSpec((_BLKW, 128), lambda i: (i, 0))


def _partw():
    return pl.BlockSpec((2, _BLKW, 128), lambda i: (0, i, 0))


def _t0_body(h0_ref, dp_ref, bd_ref, w1_ref, dinv_ref, g1_ref):
    psum = dp_ref[0] + dp_ref[1]
    deg = jnp.dot(psum, bd_ref[...], preferred_element_type=_f32) + 1.0
    dinv = lax.rsqrt(deg)
    hw = jnp.dot(h0_ref[...], w1_ref[...], preferred_element_type=_f32)
    dinv_ref[...] = dinv
    g1_ref[...] = dinv * hw


def _tc0(h0r, degp_w, bd, w1k):
    return pl.pallas_call(
        _t0_body,
        grid=(_NBLK,),
        in_specs=[pl.BlockSpec((_BLKW, 24), lambda i: (i, 0)), _partw(),
                  _full((128, 128)), _full((24, 128))],
        out_specs=[_rowsw(), _rowsw()],
        out_shape=[jax.ShapeDtypeStruct((_NW, 128), _f32),
                   jax.ShapeDtypeStruct((_NW, 128), _f32)],
    )(h0r, degp_w, bd, w1k)


def _mid_body(ns_in, ns_out, refs):
    parts = refs[:ns_in]
    gs = refs[ns_in:2 * ns_in]
    dinv_ref, w_ref, b_ref = refs[2 * ns_in:2 * ns_in + 3]
    outs = refs[2 * ns_in + 3:]
    dinv = dinv_ref[...]
    b = b_ref[...]
    cols = []
    for s in range(ns_in):
        p = parts[s]
        cols.append(dinv * (p[0] + p[1] + gs[s][...])
                    + b[:, s * 128:(s + 1) * 128])
    z = jnp.maximum(jnp.concatenate(cols, axis=1) if ns_in > 1 else cols[0],
                    0.0)
    r = jnp.dot(z, w_ref[...], preferred_element_type=_f32)
    for t in range(ns_out):
        outs[t][...] = dinv * r[:, t * 128:(t + 1) * 128]


def _tc_mid(parts, gs, dinv, wk, bt):
    ns_in = len(gs)
    ns_out = wk.shape[1] // 128
    body = functools.partial(_mid_body, ns_in, ns_out)

    def wrapped(*refs):
        body(refs)
    return pl.pallas_call(
        wrapped,
        grid=(_NBLK,),
        in_specs=([_partw()] * ns_in + [_rowsw()] * ns_in
                  + [_rowsw(), _full(wk.shape), _full((1, ns_in * 128))]),
        out_specs=[_rowsw()] * ns_out,
        out_shape=[jax.ShapeDtypeStruct((_NW, 128), _f32)] * ns_out,
    )(*parts, *gs, dinv, wk, bt)


def _t4_body(p_ref, g_ref, dinv_ref, b_ref, out_ref):
    h = dinv_ref[...] * (p_ref[0] + p_ref[1] + g_ref[...]) + b_ref[...]
    lane = lax.broadcasted_iota(jnp.int32, (_BLKW, 128), 1)
    out_ref[...] = jnp.where(lane % L == 10, h + 1.0, h)


def _tc4(p4, g4, dinv, b4t):
    return pl.pallas_call(
        _t4_body,
        grid=(_NBLK,),
        in_specs=[_partw(), _rowsw(), _rowsw(), _full((1, 128))],
        out_specs=_rowsw(),
        out_shape=jax.ShapeDtypeStruct((NPOOL // 8, 128), _f32),
    )(p4, g4, dinv, b4t)


def _kron8(wblk):
    return jnp.kron(jnp.eye(8, dtype=_f32), wblk)


def _big_w(w):
    si, so = w.shape[0] // L, w.shape[1] // L
    return jnp.concatenate([
        jnp.concatenate([_kron8(w[s * L:(s + 1) * L, t * L:(t + 1) * L])
                         for t in range(so)], axis=1)
        for s in range(si)], axis=0)


def _bias_tile(b):
    return jnp.concatenate(
        [jnp.tile(b[s * L:(s + 1) * L], 8) for s in range(b.shape[0] // L)]
    ).reshape(1, -1)


_edge_pass = functools.lru_cache(maxsize=None)(_make_edge_pass)


def kernel(x, pos, edge_index, batch, W1, b1, W2, b2, W3, b3, W4, b4):
    npad = EPAD - E
    src_m = edge_index[0].reshape(EROWS_MAIN, 128)
    dst_m = edge_index[1].reshape(EROWS_MAIN, 128)
    ar = jnp.arange(npad, dtype=jnp.int32)
    src_p = ((ar * 2003) % N).reshape(npad // 128, 128)
    dst_p = (N + ar % (NACC - N)).reshape(npad // 128, 128)
    bpad = jnp.concatenate(
        [batch, G + (jnp.arange(NPOOL - N, dtype=jnp.int32) % (GACC - G))])
    b2d = bpad.reshape(NPOOL // 128, 128)

    def wide(p):
        return p.reshape(2, _NACCW, 128)

    def tbl(gw):
        return gw.reshape(NACC, L)

    h0r = jnp.pad(jnp.concatenate([x, pos], axis=1).reshape(N // 8, 24),
                  ((0, _NW - N // 8), (0, 0)))
    ones_row = jnp.zeros((L, L), _f32).at[0, :].set(1.0)
    bd = _kron8(ones_row)
    w1k = _kron8(W1)

    (degp,) = _edge_pass(1, False)(dst_m, dst_p)
    dinv, g1 = _tc0(h0r, wide(degp), bd, w1k)
    (p1,) = _edge_pass(1, True)(dst_m, dst_p, src_m, src_p, tbl(g1))
    g2 = _tc_mid([wide(p1)], [g1], dinv, _big_w(W2), _bias_tile(b1))
    p2 = _edge_pass(2, True)(dst_m, dst_p, src_m, src_p,
                             tbl(g2[0]), tbl(g2[1]))
    g3 = _tc_mid([wide(p) for p in p2], list(g2), dinv, _big_w(W3),
                 _bias_tile(b2))
    p3 = _edge_pass(4, True)(dst_m, dst_p, src_m, src_p, tbl(g3[0]),
                             tbl(g3[1]), tbl(g3[2]), tbl(g3[3]))
    W4p = jnp.pad(W4, ((0, 0), (0, L - W4.shape[1])))
    (g4,) = _tc_mid([wide(p) for p in p3], list(g3), dinv, _big_w(W4p),
                    _bias_tile(b3))
    (p4,) = _edge_pass(1, True)(dst_m, dst_p, src_m, src_p, tbl(g4))
    b4t = _bias_tile(jnp.pad(b4, (0, L - b4.shape[0])))
    h4p = _tc4(wide(p4), g4, dinv, b4t)
    pool = _pool_pass()(h4p.reshape(NPOOL, L), b2d)
    tot = pool[0] + pool[1]
    return tot[:, :10] / jnp.maximum(tot[:, 10:11], 1.0)


# speedup vs baseline: 38.0732x; 1.1066x over previous
import functools

import jax
import jax.numpy as jnp
from jax import lax
from jax.experimental import pallas as pl
from jax.experimental.pallas import tpu as pltpu
from jax.experimental.pallas import tpu_sc as plsc

N = 100000
E = 1600000
G = 256
L = 16

NUM_TILES = 32
CHUNK = 512
KSUB = CHUNK // 128
EPAD = 1605632
TILE_CHUNKS = EPAD // NUM_TILES // CHUNK
NPAIR = TILE_CHUNKS // 2
EROWS_PER_TILE = EPAD // NUM_TILES // 128
EROWS_MAIN = E // 128

NACC = 102400
ROWS_PER_TILE = NACC // 16
ZCH = 320

NPOOL = 102400
POOL_CHUNK = 640
GACC = G + 32

_f32 = jnp.float32


@functools.lru_cache(maxsize=None)
def _mesh():
    return plsc.VectorSubcoreMesh(core_axis_name="c", subcore_axis_name="s")


def _wid(cid, sid):
    return sid * 2 + cid


def _fill_zero(zref, nrows):
    def zrow(i, _):
        zref[i, :] = jnp.zeros((L,), _f32)
        return 0
    lax.fori_loop(0, nrows, zrow, 0)


def _zero_acc(acc, zero_v, sid, rows_per_tile, zch):
    def zchunk(i, _):
        pltpu.sync_copy(zero_v, acc.at[pl.ds(sid * rows_per_tile + i * zch, zch)])
        return 0
    lax.fori_loop(0, rows_per_tile // zch, zchunk, 0)


def _edge_scatter_pass(g_hbm, edges_m, edges_p, src_v, dst_v, rows_v, acc,
                       sem_g, sem_s, wid):
    base = wid * EROWS_PER_TILE
    gather = g_hbm is not None

    def load_idx(c, x):
        row0 = base + c * KSUB

        @pl.when(row0 < EROWS_MAIN)
        def _():
            if gather:
                pltpu.sync_copy(edges_m.at[0].at[pl.ds(row0, KSUB)], src_v[x])
            pltpu.sync_copy(edges_m.at[1].at[pl.ds(row0, KSUB)], dst_v[x])

        @pl.when(row0 >= EROWS_MAIN)
        def _():
            if gather:
                pltpu.sync_copy(edges_p.at[0].at[pl.ds(row0 - EROWS_MAIN,
                                                       KSUB)], src_v[x])
            pltpu.sync_copy(edges_p.at[1].at[pl.ds(row0 - EROWS_MAIN, KSUB)],
                            dst_v[x])

    def fire_gather(x):
        for j in range(KSUB):
            pltpu.async_copy(g_hbm.at[src_v[x].at[j]],
                             rows_v[x].at[pl.ds(j * 128, 128)], sem_g[x])

    def drain_gather(x):
        pltpu.make_async_copy(g_hbm.at[pl.ds(0, CHUNK)], rows_v[x],
                              sem_g[x]).wait()

    def fire_scatter(x):
        r = rows_v[x] if g_hbm is not None else rows_v[0]
        for j in range(KSUB):
            pltpu.async_copy(r.at[pl.ds(j * 128, 128)],
                             acc.at[dst_v[x].at[j]], sem_s[x], add=True)

    def drain_scatter(x):
        pltpu.make_async_copy(rows_v[x] if g_hbm is not None else rows_v[0],
                              acc.at[pl.ds(0, CHUNK)], sem_s[x]).wait()

    if g_hbm is None:
        def dpair(p, _):
            for x in range(2):
                @pl.when(p > 0)
                def _():
                    drain_scatter(x)
                load_idx(2 * p + x, x)
                fire_scatter(x)
            return 0
        lax.fori_loop(0, NPAIR, dpair, 0)
        drain_scatter(0)
        drain_scatter(1)
        return

    load_idx(0, 0)
    fire_gather(0)

    def pipe(p, _):
        @pl.when(p > 0)
        def _():
            drain_scatter(1)
        load_idx(2 * p + 1, 1)
        fire_gather(1)
        drain_gather(0)
        fire_scatter(0)

        @pl.when(p < NPAIR - 1)
        def _():
            drain_scatter(0)
            load_idx(2 * p + 2, 0)
            fire_gather(0)
        drain_gather(1)
        fire_scatter(1)
        return 0
    lax.fori_loop(0, NPAIR, pipe, 0)
    drain_scatter(0)
    drain_scatter(1)


def _make_edge_pass(num_slices, gather):
    n_in = 2 + (num_slices if gather else 0)

    def body(*refs):
        ins = refs[:n_in]
        outs = refs[n_in:n_in + num_slices]
        (s0, s1, d0, d1, r0, r1, zero_v, acc,
         sg0, sg1, ss0, ss1) = refs[n_in + num_slices:]
        src_v, dst_v, rows_v = (s0, s1), (d0, d1), (r0, r1)
        sem_g, sem_s = (sg0, sg1), (ss0, ss1)
        edges_m, edges_p = ins[0], ins[1]
        cid = lax.axis_index("c")
        sid = lax.axis_index("s")
        wid = _wid(cid, sid)
        _fill_zero(zero_v, ZCH)
        if not gather:
            def orow(i, _):
                r0[i, :] = jnp.ones((L,), _f32)
                return 0
            lax.fori_loop(0, CHUNK, orow, 0)
        for s in range(num_slices):
            g_hbm = ins[2 + s] if gather else None
            _zero_acc(acc, zero_v, sid, ROWS_PER_TILE, ZCH)
            plsc.subcore_barrier()
            _edge_scatter_pass(g_hbm, edges_m, edges_p, src_v, dst_v, rows_v,
                               acc, sem_g, sem_s, wid)
            plsc.subcore_barrier()
            pltpu.sync_copy(
                acc.at[pl.ds(sid * ROWS_PER_TILE, ROWS_PER_TILE)],
                outs[s].at[cid].at[pl.ds(sid * ROWS_PER_TILE, ROWS_PER_TILE)])
            if s < num_slices - 1:
                plsc.subcore_barrier()

    out_type = [jax.ShapeDtypeStruct((2, NACC, L), _f32)] * num_slices
    scratch = [
        pltpu.VMEM((KSUB, 128), jnp.int32),
        pltpu.VMEM((KSUB, 128), jnp.int32),
        pltpu.VMEM((KSUB, 128), jnp.int32),
        pltpu.VMEM((KSUB, 128), jnp.int32),
        pltpu.VMEM((CHUNK, L), _f32),
        pltpu.VMEM((CHUNK, L), _f32),
        pltpu.VMEM((ZCH, L), _f32),
        pltpu.VMEM_SHARED((NACC, L), _f32),
        pltpu.SemaphoreType.DMA,
        pltpu.SemaphoreType.DMA,
        pltpu.SemaphoreType.DMA,
        pltpu.SemaphoreType.DMA,
    ]
    return pl.kernel(
        body, out_type=out_type, mesh=_mesh(), scratch_types=scratch,
        compiler_params=pltpu.CompilerParams(use_tc_tiling_on_sc=False))


def _pool_body(h_hbm, b2d_hbm, out_hbm, bat_v, rows_v, zero_v, acc, sem):
    del sem
    cid = lax.axis_index("c")
    sid = lax.axis_index("s")
    wid = _wid(cid, sid)
    _fill_zero(zero_v, GACC // 16)
    pltpu.sync_copy(zero_v, acc.at[pl.ds(sid * (GACC // 16), GACC // 16)])
    plsc.subcore_barrier()
    base = wid * (NPOOL // NUM_TILES)
    brow0 = wid * (NPOOL // NUM_TILES // 128)

    def pchunk(c, _):
        pltpu.sync_copy(h_hbm.at[pl.ds(base + c * POOL_CHUNK, POOL_CHUNK)], rows_v)
        pltpu.sync_copy(b2d_hbm.at[pl.ds(brow0 + c * (POOL_CHUNK // 128),
                                         POOL_CHUNK // 128)], bat_v)
        def sfire(j, _):
            pltpu.sync_copy(rows_v.at[pl.ds(j * 128, 128)],
                            acc.at[bat_v.at[j]], add=True)
            return 0
        lax.fori_loop(0, POOL_CHUNK // 128, sfire, 0)
        return 0
    lax.fori_loop(0, NPOOL // NUM_TILES // POOL_CHUNK, pchunk, 0)
    plsc.subcore_barrier()
    pltpu.sync_copy(acc.at[pl.ds(sid * (G // 16), G // 16)],
                    out_hbm.at[cid].at[pl.ds(sid * (G // 16), G // 16)])


@functools.lru_cache(maxsize=None)
def _pool_pass():
    return pl.kernel(
        _pool_body,
        out_type=jax.ShapeDtypeStruct((2, G, L), _f32),
        mesh=_mesh(),
        scratch_types=[
            pltpu.VMEM((POOL_CHUNK // 128, 128), jnp.int32),
            pltpu.VMEM((POOL_CHUNK, L), _f32),
            pltpu.VMEM((GACC // 16, L), _f32),
            pltpu.VMEM_SHARED((GACC, L), _f32),
            pltpu.SemaphoreType.DMA,
        ],
        compiler_params=pltpu.CompilerParams(use_tc_tiling_on_sc=False))


_BLKW = 256
_NW = NACC // 8
_NBLK = _NW // _BLKW
_NACCW = NACC // 8


def _full(shape):
    return pl.BlockSpec(shape, lambda i: (0,) * len(shape))


def _rowsw():
    return pl.BlockSpec((_BLKW, 128), lambda i: (i, 0))


def _partw():
    return pl.BlockSpec((2, _BLKW, 128), lambda i: (0, i, 0))


def _t0_body(h0_ref, dp_ref, bd_ref, w1_ref, dinv_ref, g1_ref):
    psum = dp_ref[0] + dp_ref[1]
    deg = jnp.dot(psum, bd_ref[...], preferred_element_type=_f32) + 1.0
    dinv = lax.rsqrt(deg)
    hw = jnp.dot(h0_ref[...], w1_ref[...], preferred_element_type=_f32)
    dinv_ref[...] = dinv
    g1_ref[...] = dinv * hw


def _tc0(h0r, degp_w, bd, w1k):
    return pl.pallas_call(
        _t0_body,
        grid=(_NBLK,),
        in_specs=[pl.BlockSpec((_BLKW, 24), lambda i: (i, 0)), _partw(),
                  _full((128, 128)), _full((24, 128))],
        out_specs=[_rowsw(), _rowsw()],
        out_shape=[jax.ShapeDtypeStruct((_NW, 128), _f32),
                   jax.ShapeDtypeStruct((_NW, 128), _f32)],
    )(h0r, degp_w, bd, w1k)


def _mid_body(ns_in, ns_out, refs):
    parts = refs[:ns_in]
    gs = refs[ns_in:2 * ns_in]
    dinv_ref, w_ref, b_ref = refs[2 * ns_in:2 * ns_in + 3]
    outs = refs[2 * ns_in + 3:]
    dinv = dinv_ref[...]
    b = b_ref[...]
    cols = []
    for s in range(ns_in):
        p = parts[s]
        cols.append(dinv * (p[0] + p[1] + gs[s][...])
                    + b[:, s * 128:(s + 1) * 128])
    z = jnp.maximum(jnp.concatenate(cols, axis=1) if ns_in > 1 else cols[0],
                    0.0)
    r = jnp.dot(z, w_ref[...], preferred_element_type=_f32)
    for t in range(ns_out):
        outs[t][...] = dinv * r[:, t * 128:(t + 1) * 128]


def _tc_mid(parts, gs, dinv, wk, bt):
    ns_in = len(gs)
    ns_out = wk.shape[1] // 128
    body = functools.partial(_mid_body, ns_in, ns_out)

    def wrapped(*refs):
        body(refs)
    return pl.pallas_call(
        wrapped,
        grid=(_NBLK,),
        in_specs=([_partw()] * ns_in + [_rowsw()] * ns_in
                  + [_rowsw(), _full(wk.shape), _full((1, ns_in * 128))]),
        out_specs=[_rowsw()] * ns_out,
        out_shape=[jax.ShapeDtypeStruct((_NW, 128), _f32)] * ns_out,
    )(*parts, *gs, dinv, wk, bt)


def _t4_body(p_ref, g_ref, dinv_ref, b_ref, out_ref):
    h = dinv_ref[...] * (p_ref[0] + p_ref[1] + g_ref[...]) + b_ref[...]
    lane = lax.broadcasted_iota(jnp.int32, (_BLKW, 128), 1)
    out_ref[...] = jnp.where(lane % L == 10, h + 1.0, h)


def _tc4(p4, g4, dinv, b4t):
    return pl.pallas_call(
        _t4_body,
        grid=(_NBLK,),
        in_specs=[_partw(), _rowsw(), _rowsw(), _full((1, 128))],
        out_specs=_rowsw(),
        out_shape=jax.ShapeDtypeStruct((NPOOL // 8, 128), _f32),
    )(p4, g4, dinv, b4t)


def _kron8(wblk):
    return jnp.kron(jnp.eye(8, dtype=_f32), wblk)


def _big_w(w):
    si, so = w.shape[0] // L, w.shape[1] // L
    return jnp.concatenate([
        jnp.concatenate([_kron8(w[s * L:(s + 1) * L, t * L:(t + 1) * L])
                         for t in range(so)], axis=1)
        for s in range(si)], axis=0)


def _bias_tile(b):
    return jnp.concatenate(
        [jnp.tile(b[s * L:(s + 1) * L], 8) for s in range(b.shape[0] // L)]
    ).reshape(1, -1)


_edge_pass = functools.lru_cache(maxsize=None)(_make_edge_pass)


def kernel(x, pos, edge_index, batch, W1, b1, W2, b2, W3, b3, W4, b4):
    npad = EPAD - E
    edges_m = edge_index.reshape(2, EROWS_MAIN, 128)
    ar = jnp.arange(npad, dtype=jnp.int32)
    edges_p = jnp.stack([(ar * 2003) % N,
                         N + ar % (NACC - N)]).reshape(2, npad // 128, 128)
    bpad = jnp.concatenate(
        [batch, G + (jnp.arange(NPOOL - N, dtype=jnp.int32) % (GACC - G))])
    b2d = bpad.reshape(NPOOL // 128, 128)

    def wide(p):
        return p.reshape(2, _NACCW, 128)

    def tbl(gw):
        return gw.reshape(NACC, L)

    h0r = jnp.pad(jnp.concatenate([x, pos], axis=1).reshape(N // 8, 24),
                  ((0, _NW - N // 8), (0, 0)))
    ones_row = jnp.zeros((L, L), _f32).at[0, :].set(1.0)
    bd = _kron8(ones_row)
    w1k = _kron8(W1)

    (degp,) = _edge_pass(1, False)(edges_m, edges_p)
    dinv, g1 = _tc0(h0r, wide(degp), bd, w1k)
    (p1,) = _edge_pass(1, True)(edges_m, edges_p, tbl(g1))
    g2 = _tc_mid([wide(p1)], [g1], dinv, _big_w(W2), _bias_tile(b1))
    p2 = _edge_pass(2, True)(edges_m, edges_p, tbl(g2[0]), tbl(g2[1]))
    g3 = _tc_mid([wide(p) for p in p2], list(g2), dinv, _big_w(W3),
                 _bias_tile(b2))
    p3 = _edge_pass(4, True)(edges_m, edges_p, tbl(g3[0]), tbl(g3[1]),
                             tbl(g3[2]), tbl(g3[3]))
    W4p = jnp.pad(W4, ((0, 0), (0, L - W4.shape[1])))
    (g4,) = _tc_mid([wide(p) for p in p3], list(g3), dinv, _big_w(W4p),
                    _bias_tile(b3))
    (p4,) = _edge_pass(1, True)(edges_m, edges_p, tbl(g4))
    b4t = _bias_tile(jnp.pad(b4, (0, L - b4.shape[0])))
    h4p = _tc4(wide(p4), g4, dinv, b4t)
    pool = _pool_pass()(h4p.reshape(NPOOL, L), b2d)
    tot = pool[0] + pool[1]
    return tot[:, :10] / jnp.maximum(tot[:, 10:11], 1.0)


# speedup vs baseline: 41.7562x; 1.0967x over previous
import functools

import jax
import jax.numpy as jnp
from jax import lax
from jax.experimental import pallas as pl
from jax.experimental.pallas import tpu as pltpu
from jax.experimental.pallas import tpu_sc as plsc

N = 100000
E = 1600000
G = 256
L = 16

NUM_TILES = 32
CHUNK = 640
KSUB = CHUNK // 128
EPAD = 1638400
TILE_CHUNKS = EPAD // NUM_TILES // CHUNK
NPAIR = TILE_CHUNKS // 2
EROWS_PER_TILE = EPAD // NUM_TILES // 128
EROWS_MAIN = E // 128

NACC = 102400
ROWS_PER_TILE = NACC // 16
ZCH = 320

NPOOL = 102400
POOL_CHUNK = 640
GACC = G + 32

_f32 = jnp.float32


@functools.lru_cache(maxsize=None)
def _mesh():
    return plsc.VectorSubcoreMesh(core_axis_name="c", subcore_axis_name="s")


def _wid(cid, sid):
    return sid * 2 + cid


def _fill_zero(zref, nrows):
    def zrow(i, _):
        zref[i, :] = jnp.zeros((L,), _f32)
        return 0
    lax.fori_loop(0, nrows, zrow, 0)


def _zero_acc(acc, zero_v, sid, rows_per_tile, zch):
    def zchunk(i, _):
        pltpu.sync_copy(zero_v, acc.at[pl.ds(sid * rows_per_tile + i * zch, zch)])
        return 0
    lax.fori_loop(0, rows_per_tile // zch, zchunk, 0)


def _edge_scatter_pass(g_hbm, edges_m, edges_p, src_v, dst_v, rows_v, acc,
                       sem_g, sem_s, wid):
    base = wid * EROWS_PER_TILE
    gather = g_hbm is not None

    def load_idx(c, x):
        row0 = base + c * KSUB

        @pl.when(row0 < EROWS_MAIN)
        def _():
            if gather:
                pltpu.sync_copy(edges_m.at[0].at[pl.ds(row0, KSUB)], src_v[x])
            pltpu.sync_copy(edges_m.at[1].at[pl.ds(row0, KSUB)], dst_v[x])

        @pl.when(row0 >= EROWS_MAIN)
        def _():
            if gather:
                pltpu.sync_copy(edges_p.at[0].at[pl.ds(row0 - EROWS_MAIN,
                                                       KSUB)], src_v[x])
            pltpu.sync_copy(edges_p.at[1].at[pl.ds(row0 - EROWS_MAIN, KSUB)],
                            dst_v[x])

    def fire_gather(x):
        for j in range(KSUB):
            pltpu.async_copy(g_hbm.at[src_v[x].at[j]],
                             rows_v[x].at[pl.ds(j * 128, 128)], sem_g[x])

    def drain_gather(x):
        pltpu.make_async_copy(g_hbm.at[pl.ds(0, CHUNK)], rows_v[x],
                              sem_g[x]).wait()

    def fire_scatter(x):
        r = rows_v[x] if g_hbm is not None else rows_v[0]
        for j in range(KSUB):
            pltpu.async_copy(r.at[pl.ds(j * 128, 128)],
                             acc.at[dst_v[x].at[j]], sem_s[x], add=True)

    def drain_scatter(x):
        pltpu.make_async_copy(rows_v[x] if g_hbm is not None else rows_v[0],
                              acc.at[pl.ds(0, CHUNK)], sem_s[x]).wait()

    if g_hbm is None:
        def dpair(p, _):
            for x in range(2):
                @pl.when(p > 0)
                def _():
                    drain_scatter(x)
                load_idx(2 * p + x, x)
                fire_scatter(x)
            return 0
        lax.fori_loop(0, NPAIR, dpair, 0)
        drain_scatter(0)
        drain_scatter(1)
        return

    load_idx(0, 0)
    fire_gather(0)

    def pipe(p, _):
        @pl.when(p > 0)
        def _():
            drain_scatter(1)
        load_idx(2 * p + 1, 1)
        fire_gather(1)
        drain_gather(0)
        fire_scatter(0)

        @pl.when(p < NPAIR - 1)
        def _():
            drain_scatter(0)
            load_idx(2 * p + 2, 0)
            fire_gather(0)
        drain_gather(1)
        fire_scatter(1)
        return 0
    lax.fori_loop(0, NPAIR, pipe, 0)
    drain_scatter(0)
    drain_scatter(1)


def _make_edge_pass(num_slices, gather):
    n_in = 2 + (num_slices if gather else 0)

    def body(*refs):
        ins = refs[:n_in]
        outs = refs[n_in:n_in + num_slices]
        (s0, s1, d0, d1, r0, r1, zero_v, acc,
         sg0, sg1, ss0, ss1) = refs[n_in + num_slices:]
        src_v, dst_v, rows_v = (s0, s1), (d0, d1), (r0, r1)
        sem_g, sem_s = (sg0, sg1), (ss0, ss1)
        edges_m, edges_p = ins[0], ins[1]
        cid = lax.axis_index("c")
        sid = lax.axis_index("s")
        wid = _wid(cid, sid)
        _fill_zero(zero_v, ZCH)
        if not gather:
            def orow(i, _):
                r0[i, :] = jnp.ones((L,), _f32)
                return 0
            lax.fori_loop(0, CHUNK, orow, 0)
        for s in range(num_slices):
            g_hbm = ins[2 + s] if gather else None
            _zero_acc(acc, zero_v, sid, ROWS_PER_TILE, ZCH)
            plsc.subcore_barrier()
            _edge_scatter_pass(g_hbm, edges_m, edges_p, src_v, dst_v, rows_v,
                               acc, sem_g, sem_s, wid)
            plsc.subcore_barrier()
            pltpu.sync_copy(
                acc.at[pl.ds(sid * ROWS_PER_TILE, ROWS_PER_TILE)],
                outs[s].at[cid].at[pl.ds(sid * ROWS_PER_TILE, ROWS_PER_TILE)])
            if s < num_slices - 1:
                plsc.subcore_barrier()

    out_type = [jax.ShapeDtypeStruct((2, NACC, L), _f32)] * num_slices
    scratch = [
        pltpu.VMEM((KSUB, 128), jnp.int32),
        pltpu.VMEM((KSUB, 128), jnp.int32),
        pltpu.VMEM((KSUB, 128), jnp.int32),
        pltpu.VMEM((KSUB, 128), jnp.int32),
        pltpu.VMEM((CHUNK, L), _f32),
        pltpu.VMEM((CHUNK, L), _f32),
        pltpu.VMEM((ZCH, L), _f32),
        pltpu.VMEM_SHARED((NACC, L), _f32),
        pltpu.SemaphoreType.DMA,
        pltpu.SemaphoreType.DMA,
        pltpu.SemaphoreType.DMA,
        pltpu.SemaphoreType.DMA,
    ]
    return pl.kernel(
        body, out_type=out_type, mesh=_mesh(), scratch_types=scratch,
        compiler_params=pltpu.CompilerParams(use_tc_tiling_on_sc=False))


def _pool_body(h_hbm, b2d_hbm, out_hbm, bat_v, rows_v, zero_v, acc, sem):
    del sem
    cid = lax.axis_index("c")
    sid = lax.axis_index("s")
    wid = _wid(cid, sid)
    _fill_zero(zero_v, GACC // 16)
    pltpu.sync_copy(zero_v, acc.at[pl.ds(sid * (GACC // 16), GACC // 16)])
    plsc.subcore_barrier()
    base = wid * (NPOOL // NUM_TILES)
    brow0 = wid * (NPOOL // NUM_TILES // 128)

    def pchunk(c, _):
        pltpu.sync_copy(h_hbm.at[pl.ds(base + c * POOL_CHUNK, POOL_CHUNK)], rows_v)
        pltpu.sync_copy(b2d_hbm.at[pl.ds(brow0 + c * (POOL_CHUNK // 128),
                                         POOL_CHUNK // 128)], bat_v)
        def sfire(j, _):
            pltpu.sync_copy(rows_v.at[pl.ds(j * 128, 128)],
                            acc.at[bat_v.at[j]], add=True)
            return 0
        lax.fori_loop(0, POOL_CHUNK // 128, sfire, 0)
        return 0
    lax.fori_loop(0, NPOOL // NUM_TILES // POOL_CHUNK, pchunk, 0)
    plsc.subcore_barrier()
    pltpu.sync_copy(acc.at[pl.ds(sid * (G // 16), G // 16)],
                    out_hbm.at[cid].at[pl.ds(sid * (G // 16), G // 16)])


@functools.lru_cache(maxsize=None)
def _pool_pass():
    return pl.kernel(
        _pool_body,
        out_type=jax.ShapeDtypeStruct((2, G, L), _f32),
        mesh=_mesh(),
        scratch_types=[
            pltpu.VMEM((POOL_CHUNK // 128, 128), jnp.int32),
            pltpu.VMEM((POOL_CHUNK, L), _f32),
            pltpu.VMEM((GACC // 16, L), _f32),
            pltpu.VMEM_SHARED((GACC, L), _f32),
            pltpu.SemaphoreType.DMA,
        ],
        compiler_params=pltpu.CompilerParams(use_tc_tiling_on_sc=False))


_BLKW = 256
_NW = NACC // 8
_NBLK = _NW // _BLKW
_NACCW = NACC // 8


def _full(shape):
    return pl.BlockSpec(shape, lambda i: (0,) * len(shape))


def _rowsw():
    return pl.BlockSpec((_BLKW, 128), lambda i: (i, 0))


def _partw():
    return pl.BlockSpec((2, _BLKW, 128), lambda i: (0, i, 0))


def _t0_body(h0_ref, dp_ref, bd_ref, w1_ref, dinv_ref, g1_ref):
    psum = dp_ref[0] + dp_ref[1]
    deg = jnp.dot(psum, bd_ref[...], preferred_element_type=_f32) + 1.0
    dinv = lax.rsqrt(deg)
    hw = jnp.dot(h0_ref[...], w1_ref[...], preferred_element_type=_f32)
    dinv_ref[...] = dinv
    g1_ref[...] = dinv * hw


def _tc0(h0r, degp_w, bd, w1k):
    return pl.pallas_call(
        _t0_body,
        grid=(_NBLK,),
        in_specs=[pl.BlockSpec((_BLKW, 24), lambda i: (i, 0)), _partw(),
                  _full((128, 128)), _full((24, 128))],
        out_specs=[_rowsw(), _rowsw()],
        out_shape=[jax.ShapeDtypeStruct((_NW, 128), _f32),
                   jax.ShapeDtypeStruct((_NW, 128), _f32)],
    )(h0r, degp_w, bd, w1k)


def _mid_body(ns_in, ns_out, refs):
    parts = refs[:ns_in]
    gs = refs[ns_in:2 * ns_in]
    dinv_ref, w_ref, b_ref = refs[2 * ns_in:2 * ns_in + 3]
    outs = refs[2 * ns_in + 3:]
    dinv = dinv_ref[...]
    b = b_ref[...]
    cols = []
    for s in range(ns_in):
        p = parts[s]
        cols.append(dinv * (p[0] + p[1] + gs[s][...])
                    + b[:, s * 128:(s + 1) * 128])
    z = jnp.maximum(jnp.concatenate(cols, axis=1) if ns_in > 1 else cols[0],
                    0.0)
    r = jnp.dot(z, w_ref[...], preferred_element_type=_f32)
    for t in range(ns_out):
        outs[t][...] = dinv * r[:, t * 128:(t + 1) * 128]


def _tc_mid(parts, gs, dinv, wk, bt):
    ns_in = len(gs)
    ns_out = wk.shape[1] // 128
    body = functools.partial(_mid_body, ns_in, ns_out)

    def wrapped(*refs):
        body(refs)
    return pl.pallas_call(
        wrapped,
        grid=(_NBLK,),
        in_specs=([_partw()] * ns_in + [_rowsw()] * ns_in
                  + [_rowsw(), _full(wk.shape), _full((1, ns_in * 128))]),
        out_specs=[_rowsw()] * ns_out,
        out_shape=[jax.ShapeDtypeStruct((_NW, 128), _f32)] * ns_out,
    )(*parts, *gs, dinv, wk, bt)


def _t4_body(p_ref, g_ref, dinv_ref, b_ref, out_ref):
    h = dinv_ref[...] * (p_ref[0] + p_ref[1] + g_ref[...]) + b_ref[...]
    lane = lax.broadcasted_iota(jnp.int32, (_BLKW, 128), 1)
    out_ref[...] = jnp.where(lane % L == 10, h + 1.0, h)


def _tc4(p4, g4, dinv, b4t):
    return pl.pallas_call(
        _t4_body,
        grid=(_NBLK,),
        in_specs=[_partw(), _rowsw(), _rowsw(), _full((1, 128))],
        out_specs=_rowsw(),
        out_shape=jax.ShapeDtypeStruct((NPOOL // 8, 128), _f32),
    )(p4, g4, dinv, b4t)


def _kron8(wblk):
    return jnp.kron(jnp.eye(8, dtype=_f32), wblk)


def _big_w(w):
    si, so = w.shape[0] // L, w.shape[1] // L
    return jnp.concatenate([
        jnp.concatenate([_kron8(w[s * L:(s + 1) * L, t * L:(t + 1) * L])
                         for t in range(so)], axis=1)
        for s in range(si)], axis=0)


def _bias_tile(b):
    return jnp.concatenate(
        [jnp.tile(b[s * L:(s + 1) * L], 8) for s in range(b.shape[0] // L)]
    ).reshape(1, -1)


_edge_pass = functools.lru_cache(maxsize=None)(_make_edge_pass)


def kernel(x, pos, edge_index, batch, W1, b1, W2, b2, W3, b3, W4, b4):
    npad = EPAD - E
    edges_m = edge_index.reshape(2, EROWS_MAIN, 128)
    ar = jnp.arange(npad, dtype=jnp.int32)
    edges_p = jnp.stack([(ar * 2003) % N,
                         N + ar % (NACC - N)]).reshape(2, npad // 128, 128)
    bpad = jnp.concatenate(
        [batch, G + (jnp.arange(NPOOL - N, dtype=jnp.int32) % (GACC - G))])
    b2d = bpad.reshape(NPOOL // 128, 128)

    def wide(p):
        return p.reshape(2, _NACCW, 128)

    def tbl(gw):
        return gw.reshape(NACC, L)

    h0r = jnp.pad(jnp.concatenate([x, pos], axis=1).reshape(N // 8, 24),
                  ((0, _NW - N // 8), (0, 0)))
    ones_row = jnp.zeros((L, L), _f32).at[0, :].set(1.0)
    bd = _kron8(ones_row)
    w1k = _kron8(W1)

    (degp,) = _edge_pass(1, False)(edges_m, edges_p)
    dinv, g1 = _tc0(h0r, wide(degp), bd, w1k)
    (p1,) = _edge_pass(1, True)(edges_m, edges_p, tbl(g1))
    g2 = _tc_mid([wide(p1)], [g1], dinv, _big_w(W2), _bias_tile(b1))
    p2 = _edge_pass(2, True)(edges_m, edges_p, tbl(g2[0]), tbl(g2[1]))
    g3 = _tc_mid([wide(p) for p in p2], list(g2), dinv, _big_w(W3),
                 _bias_tile(b2))
    p3 = _edge_pass(4, True)(edges_m, edges_p, tbl(g3[0]), tbl(g3[1]),
                             tbl(g3[2]), tbl(g3[3]))
    W4p = jnp.pad(W4, ((0, 0), (0, L - W4.shape[1])))
    (g4,) = _tc_mid([wide(p) for p in p3], list(g3), dinv, _big_w(W4p),
                    _bias_tile(b3))
    (p4,) = _edge_pass(1, True)(edges_m, edges_p, tbl(g4))
    b4t = _bias_tile(jnp.pad(b4, (0, L - b4.shape[0])))
    h4p = _tc4(wide(p4), g4, dinv, b4t)
    pool = _pool_pass()(h4p.reshape(NPOOL, L), b2d)
    tot = pool[0] + pool[1]
    return tot[:, :10] / jnp.maximum(tot[:, 10:11], 1.0)
